# baseline (speedup 1.0000x reference)
import jax
import jax.numpy as jnp
from jax import lax
from jax.experimental import pallas as pl
from jax.experimental.pallas import tpu as pltpu

B, C, H, W = 4, 384, 224, 224
ROW = H * W
R = B * C


def _tc_body(in_hbm, out_hbm, sem):
    @pl.loop(0, R)
    def _(r):
        src = 2 * (r // C) * C + (C - 1) - r
        pltpu.async_copy(in_hbm.at[src], out_hbm.at[r], sem)

    pltpu.make_async_copy(in_hbm, out_hbm, sem).wait()


_tc_kernel = pl.pallas_call(
    _tc_body,
    out_shape=jax.ShapeDtypeStruct((R, ROW), jnp.float32),
    in_specs=[pl.BlockSpec(memory_space=pltpu.HBM)],
    out_specs=pl.BlockSpec(memory_space=pltpu.HBM),
    scratch_shapes=[pltpu.SemaphoreType.DMA],
)


@jax.jit
def kernel(input):
    flat = input.reshape(R, ROW)
    out = _tc_kernel(flat)
    return out.reshape(B, C, H, W)


# speedup vs baseline: 7.7042x; 7.7042x over previous
import jax
import jax.numpy as jnp
from jax import lax
from jax.experimental import pallas as pl
from jax.experimental.pallas import tpu as pltpu
from jax.experimental.pallas import tpu_sc as plsc

B, C, H, W = 4, 384, 224, 224
ROW = H * W
R = B * C

_info = plsc.get_sparse_core_info()
_NC = _info.num_cores
_NS = _info.num_subcores
NW = _NC * _NS
RPW = R // NW


def _sc_body(in_hbm, out_hbm, buf0, buf1, sem0, sem1):
    wid = lax.axis_index("s") * _NC + lax.axis_index("c")
    base = wid * RPW
    b = base // C
    src0 = 2 * b * C + (C - 1) - base

    pltpu.async_copy(in_hbm.at[src0], buf0, sem0)
    pltpu.async_copy(in_hbm.at[src0 - 1], buf1, sem1)

    @pl.loop(0, RPW, step=2)
    def _(g):
        pltpu.make_async_copy(in_hbm.at[src0 - g], buf0, sem0).wait()
        pltpu.sync_copy(buf0, out_hbm.at[base + g])

        @pl.when(g + 2 < RPW)
        def _():
            pltpu.async_copy(in_hbm.at[src0 - (g + 2)], buf0, sem0)

        pltpu.make_async_copy(in_hbm.at[src0 - (g + 1)], buf1, sem1).wait()
        pltpu.sync_copy(buf1, out_hbm.at[base + g + 1])

        @pl.when(g + 3 < RPW)
        def _():
            pltpu.async_copy(in_hbm.at[src0 - (g + 3)], buf1, sem1)


_sc_kernel = pl.kernel(
    _sc_body,
    out_type=jax.ShapeDtypeStruct((R, ROW), jnp.float32),
    mesh=plsc.VectorSubcoreMesh(core_axis_name="c", subcore_axis_name="s"),
    scratch_types=[
        pltpu.VMEM((ROW,), jnp.float32),
        pltpu.VMEM((ROW,), jnp.float32),
        pltpu.SemaphoreType.DMA,
        pltpu.SemaphoreType.DMA,
    ],
)


@jax.jit
def kernel(input):
    flat = input.reshape(R, ROW)
    out = _sc_kernel(flat)
    return out.reshape(B, C, H, W)


# speedup vs baseline: 11.8641x; 1.5400x over previous
import jax
import jax.numpy as jnp
from jax import lax
from jax.experimental import pallas as pl
from jax.experimental.pallas import tpu as pltpu
from jax.experimental.pallas import tpu_sc as plsc

B, C, H, W = 4, 384, 224, 224

_info = plsc.get_sparse_core_info()
_NC = _info.num_cores
_NS = _info.num_subcores
NW = _NC * _NS
PPW = (B * C) // NW


def _sc_body(in_hbm, out_hbm, buf0, buf1, sem0, sem1):
    wid = lax.axis_index("s") * _NC + lax.axis_index("c")
    base = wid * PPW
    b = base // C
    c0 = base - b * C
    s0 = C - 1 - c0

    pltpu.async_copy(in_hbm.at[b, s0], buf0, sem0)
    pltpu.async_copy(in_hbm.at[b, s0 - 1], buf1, sem1)

    @pl.loop(0, PPW, step=2)
    def _(g):
        pltpu.make_async_copy(in_hbm.at[b, s0 - g], buf0, sem0).wait()
        pltpu.sync_copy(buf0, out_hbm.at[b, c0 + g])

        @pl.when(g + 2 < PPW)
        def _():
            pltpu.async_copy(in_hbm.at[b, s0 - (g + 2)], buf0, sem0)

        pltpu.make_async_copy(in_hbm.at[b, s0 - (g + 1)], buf1, sem1).wait()
        pltpu.sync_copy(buf1, out_hbm.at[b, c0 + g + 1])

        @pl.when(g + 3 < PPW)
        def _():
            pltpu.async_copy(in_hbm.at[b, s0 - (g + 3)], buf1, sem1)


_sc_kernel = pl.kernel(
    _sc_body,
    out_type=jax.ShapeDtypeStruct((B, C, H, W), jnp.float32),
    mesh=plsc.VectorSubcoreMesh(core_axis_name="c", subcore_axis_name="s"),
    scratch_types=[
        pltpu.VMEM((H, W), jnp.float32),
        pltpu.VMEM((H, W), jnp.float32),
        pltpu.SemaphoreType.DMA,
        pltpu.SemaphoreType.DMA,
    ],
)


@jax.jit
def kernel(input):
    return _sc_kernel(input)


# speedup vs baseline: 11.8818x; 1.0015x over previous
import jax
import jax.numpy as jnp
from jax import lax
from jax.experimental import pallas as pl
from jax.experimental.pallas import tpu as pltpu
from jax.experimental.pallas import tpu_sc as plsc

B, C, H, W = 4, 384, 224, 224

_info = plsc.get_sparse_core_info()
_NC = _info.num_cores
_NS = _info.num_subcores
NW = _NC * _NS
PPW = (B * C) // NW


def _sc_body(in_hbm, out_hbm, buf0, buf1, sem0, sem1):
    wid = lax.axis_index("s") * _NC + lax.axis_index("c")
    base = wid * PPW
    b = base // C
    c0 = base - b * C
    s0 = C - 1 - c0

    pltpu.async_copy(in_hbm.at[b, s0], buf0, sem0)
    pltpu.async_copy(in_hbm.at[b, s0 - 1], buf1, sem1)

    @pl.loop(0, PPW, step=2)
    def _(g):
        pltpu.make_async_copy(in_hbm.at[b, s0 - g], buf0, sem0).wait()
        pltpu.sync_copy(buf0, out_hbm.at[b, c0 + g])

        @pl.when(g + 2 < PPW)
        def _():
            pltpu.async_copy(in_hbm.at[b, s0 - (g + 2)], buf0, sem0)

        pltpu.make_async_copy(in_hbm.at[b, s0 - (g + 1)], buf1, sem1).wait()
        pltpu.sync_copy(buf1, out_hbm.at[b, c0 + g + 1])

        @pl.when(g + 3 < PPW)
        def _():
            pltpu.async_copy(in_hbm.at[b, s0 - (g + 3)], buf1, sem1)


_sc_kernel = pl.kernel(
    _sc_body,
    out_type=jax.ShapeDtypeStruct((B, C, H, W), jnp.float32),
    mesh=plsc.VectorSubcoreMesh(core_axis_name="c", subcore_axis_name="s"),
    compiler_params=pltpu.CompilerParams(use_tc_tiling_on_sc=True),
    scratch_types=[
        pltpu.VMEM((H, W), jnp.float32),
        pltpu.VMEM((H, W), jnp.float32),
        pltpu.SemaphoreType.DMA,
        pltpu.SemaphoreType.DMA,
    ],
)


@jax.jit
def kernel(input):
    return _sc_kernel(input)


# speedup vs baseline: 45.1197x; 3.7974x over previous
import jax
import jax.numpy as jnp
from jax import lax
from jax.experimental import pallas as pl
from jax.experimental.pallas import tpu as pltpu
from jax.experimental.pallas import tpu_sc as plsc

B, C, H, W = 4, 384, 224, 224
P = B * H * W
NG = C // 16

_info = plsc.get_sparse_core_info()
_NC = _info.num_cores
_NS = _info.num_subcores
NW = _NC * _NS
PPW = P // NW
PCH = 64
NCHUNK = PPW // PCH


def _sc_body(in_hbm, out_hbm, bin0, bin1, bout0, bout1,
             gsem0, gsem1, ssem0, ssem1):
    bins = (bin0, bin1)
    bouts = (bout0, bout1)
    gsem = (gsem0, gsem1)
    ssem = (ssem0, ssem1)

    wid = lax.axis_index("s") * _NC + lax.axis_index("c")
    pix0 = wid * PPW

    def gather(t, slot):
        pltpu.async_copy(in_hbm.at[pl.ds(pix0 + t * PCH, PCH)], bins[slot],
                         gsem[slot])

    gather(0, 0)
    gather(1, 1)

    @pl.loop(0, NCHUNK, step=2)
    def _(t0):
        for slot in range(2):
            t = t0 + slot
            src = in_hbm.at[pl.ds(pix0 + t * PCH, PCH)]
            dst = out_hbm.at[pl.ds(pix0 + t * PCH, PCH)]
            pltpu.make_async_copy(src, bins[slot], gsem[slot]).wait()

            @pl.when(t >= 2)
            def _():
                pltpu.make_async_copy(bouts[slot], dst, ssem[slot]).wait()

            @pl.loop(0, PCH)
            def _(p):
                for j in range(NG):
                    v = bins[slot][p, pl.ds(16 * (NG - 1 - j), 16)]
                    bouts[slot][p, pl.ds(16 * j, 16)] = lax.rev(v, (0,))

            pltpu.async_copy(bouts[slot], dst, ssem[slot])

            @pl.when(t + 2 < NCHUNK)
            def _():
                gather(t + 2, slot)

    for slot in range(2):
        t = NCHUNK - 2 + slot
        dst = out_hbm.at[pl.ds(pix0 + t * PCH, PCH)]
        pltpu.make_async_copy(bouts[slot], dst, ssem[slot]).wait()


_sc_kernel = pl.kernel(
    _sc_body,
    out_type=jax.ShapeDtypeStruct((P, C), jnp.float32),
    mesh=plsc.VectorSubcoreMesh(core_axis_name="c", subcore_axis_name="s"),
    scratch_types=[
        pltpu.VMEM((PCH, C), jnp.float32),
        pltpu.VMEM((PCH, C), jnp.float32),
        pltpu.VMEM((PCH, C), jnp.float32),
        pltpu.VMEM((PCH, C), jnp.float32),
        pltpu.SemaphoreType.DMA,
        pltpu.SemaphoreType.DMA,
        pltpu.SemaphoreType.DMA,
        pltpu.SemaphoreType.DMA,
    ],
)


@jax.jit
def kernel(input):
    xt = jnp.transpose(input, (0, 2, 3, 1)).reshape(P, C)
    yt = _sc_kernel(xt)
    return jnp.transpose(yt.reshape(B, H, W, C), (0, 3, 1, 2))
